# baseline (speedup 1.0000x reference)
import functools

import jax
import jax.numpy as jnp
from jax import lax
from jax.experimental import pallas as pl
from jax.experimental.pallas import tpu as pltpu
from jax.experimental.pallas import tpu_sc as plsc

_D = 32
_NC = 2
_NS = 16
_NW = _NC * _NS
_IB = 128
_CHUNK = 1024
_J = _CHUNK // _IB


def _make_body(steps):
    def body(idx_hbm, table_hbm, out_hbm, idx_v, rows_v, out_v, gsem):
        wid = lax.axis_index("s") * _NC + lax.axis_index("c")
        blk0 = wid * (steps * _J)

        def step(g, carry):
            blk = blk0 + g * _J
            pltpu.sync_copy(idx_hbm.at[pl.ds(blk, _J)], idx_v)
            for j in range(_J):
                pltpu.async_copy(
                    table_hbm.at[idx_v.at[j]],
                    rows_v.at[pl.ds(j * _IB, _IB)],
                    gsem,
                ).wait()

            evens = 2 * lax.iota(jnp.int32, 16)
            odds = evens + 1

            def row(r, c):
                rvec = jnp.full((16,), r, jnp.int32)
                a = plsc.load_gather(rows_v, [rvec, evens])
                b = plsc.load_gather(rows_v, [rvec, odds])
                out_v[r] = plsc.pack(a, b, format=plsc.PackFormat.INTERLEAVED)
                return c

            lax.fori_loop(0, _CHUNK, row, 0, unroll=4)
            pltpu.sync_copy(out_v, out_hbm.at[pl.ds(blk * _IB, _CHUNK)])
            return carry

        lax.fori_loop(0, steps, step, 0)

    return body


def kernel(inputs, embedding):
    b, s = inputs.shape
    n = b * s
    assert n % (_NW * _CHUNK) == 0
    steps = n // (_NW * _CHUNK)
    idx = inputs.reshape(-1).astype(jnp.int32).reshape(n // _IB, _IB)

    out = pl.kernel(
        _make_body(steps),
        out_type=jax.ShapeDtypeStruct((n, _D), jnp.bfloat16),
        mesh=plsc.VectorSubcoreMesh(core_axis_name="c", subcore_axis_name="s"),
        compiler_params=pltpu.CompilerParams(
            needs_layout_passes=False, use_tc_tiling_on_sc=False
        ),
        scratch_types=[
            pltpu.VMEM((_J, _IB), jnp.int32),
            pltpu.VMEM((_CHUNK, _D), jnp.float32),
            pltpu.VMEM((_CHUNK, _D), jnp.bfloat16),
            pltpu.SemaphoreType.DMA,
        ],
    )(idx, embedding)
    return out.reshape(b, s, _D)


# speedup vs baseline: 1.1491x; 1.1491x over previous
import jax
import jax.numpy as jnp
from jax import lax
from jax.experimental import pallas as pl
from jax.experimental.pallas import tpu as pltpu
from jax.experimental.pallas import tpu_sc as plsc

_D = 32
_NC = 2
_NS = 16
_NW = _NC * _NS
_IB = 128
_CHUNK = 512
_J = _CHUNK // _IB


def _make_body(steps):
    nblk = steps * _J

    def body(idx_hbm, table_hbm, out_hbm,
             idx_v, rows0, rows1, out0, out1, gsem0, gsem1, osem0, osem1):
        wid = lax.axis_index("s") * _NC + lax.axis_index("c")
        blk0 = wid * nblk
        pltpu.sync_copy(idx_hbm.at[pl.ds(blk0, nblk)], idx_v)

        evens = 2 * lax.iota(jnp.int32, 16)
        odds = evens + 1

        def fire_gather(g, rows, gsem):
            for j in range(_J):
                pltpu.async_copy(
                    table_hbm.at[idx_v.at[g * _J + j]],
                    rows.at[pl.ds(j * _IB, _IB)],
                    gsem,
                )

        def drain_gather(rows, gsem):
            pltpu.make_async_copy(
                table_hbm.at[pl.ds(0, _CHUNK)], rows, gsem
            ).wait()

        def drain_out(outb, osem):
            pltpu.make_async_copy(
                outb, out_hbm.at[pl.ds(0, _CHUNK)], osem
            ).wait()

        def compute(rows, outb):
            @plsc.parallel_loop(0, _CHUNK, unroll=8)
            def _(r):
                rvec = jnp.full((16,), r, jnp.int32)
                a = plsc.load_gather(rows, [rvec, evens])
                b = plsc.load_gather(rows, [rvec, odds])
                outb[r] = plsc.pack(a, b, format=plsc.PackFormat.INTERLEAVED)

        def make_branch(rows, outb, gsem, osem, nrows, ngsem):
            def branch(g):
                @pl.when(g + 1 < steps)
                def _():
                    fire_gather(g + 1, nrows, ngsem)

                drain_gather(rows, gsem)

                @pl.when(g >= 2)
                def _():
                    drain_out(outb, osem)

                compute(rows, outb)
                pltpu.async_copy(
                    outb,
                    out_hbm.at[pl.ds((blk0 + g * _J) * _IB, _CHUNK)],
                    osem,
                )
            return branch

        branch0 = make_branch(rows0, out0, gsem0, osem0, rows1, gsem1)
        branch1 = make_branch(rows1, out1, gsem1, osem1, rows0, gsem0)

        fire_gather(0, rows0, gsem0)

        def step(g, carry):
            pl.when(g % 2 == 0)(lambda: branch0(g))
            pl.when(g % 2 == 1)(lambda: branch1(g))
            return carry

        lax.fori_loop(0, steps, step, 0)
        drain_out(out0, osem0)
        drain_out(out1, osem1)

    return body


def kernel(inputs, embedding):
    b, s = inputs.shape
    n = b * s
    assert n % (_NW * _CHUNK) == 0
    steps = n // (_NW * _CHUNK)
    idx = inputs.reshape(-1).astype(jnp.int32).reshape(n // _IB, _IB)

    out = pl.kernel(
        _make_body(steps),
        out_type=jax.ShapeDtypeStruct((n, _D), jnp.bfloat16),
        mesh=plsc.VectorSubcoreMesh(core_axis_name="c", subcore_axis_name="s"),
        compiler_params=pltpu.CompilerParams(
            needs_layout_passes=False, use_tc_tiling_on_sc=False
        ),
        scratch_types=[
            pltpu.VMEM((steps * _J, _IB), jnp.int32),
            pltpu.VMEM((_CHUNK, _D), jnp.float32),
            pltpu.VMEM((_CHUNK, _D), jnp.float32),
            pltpu.VMEM((_CHUNK, _D), jnp.bfloat16),
            pltpu.VMEM((_CHUNK, _D), jnp.bfloat16),
            pltpu.SemaphoreType.DMA,
            pltpu.SemaphoreType.DMA,
            pltpu.SemaphoreType.DMA,
            pltpu.SemaphoreType.DMA,
        ],
    )(idx, embedding)
    return out.reshape(b, s, _D)
